# Initial kernel scaffold; baseline (speedup 1.0000x reference)
import jax
import jax.numpy as jnp
from jax.experimental import pallas as pl


def kernel(visited_node_score, visited_node_representation, rel_emb, query_src_ts_emb, query_rel_emb, Wq, Wk, Wl, bl, src, dst, query_idx):
    raise NotImplementedError("write your pallas kernel here")


# speedup vs baseline: 1.2469x; 1.2469x over previous
import functools
import jax
import jax.numpy as jnp
from jax import lax
from jax.experimental import pallas as pl

N_NODES = 50000
Q = 64
EPQ = 2048
E = Q * EPQ
D = 128
K = 128
EBLK = 512
NBLK = E // EBLK
BPQ = EPQ // EBLK

_BF = jnp.bfloat16


def _prep_body(qs_ref, qr_ref, wqc_ref, wqd_ref, wkc_ref, wkd_ref,
               lq_ref, rq_ref):
    qs = qs_ref[...].astype(_BF)
    qr = qr_ref[...].astype(_BF)
    lq_ref[...] = (jnp.dot(qs, wqc_ref[...], preferred_element_type=jnp.float32)
                   + jnp.dot(qr, wqd_ref[...], preferred_element_type=jnp.float32))
    rq_ref[...] = (jnp.dot(qs, wkc_ref[...], preferred_element_type=jnp.float32)
                   + jnp.dot(qr, wkd_ref[...], preferred_element_type=jnp.float32))


def _prep(qs, qr, wqct, wqdt, wkct, wkdt):
    outs = [jax.ShapeDtypeStruct((Q, 512), jnp.float32)] * 2
    return pl.pallas_call(_prep_body, out_shape=outs)(
        qs, qr, wqct, wqdt, wkct, wkdt)


def _logits_body(rs_ref, rd_ref, rl_ref, wqa_ref, wqb_ref, wka_ref, wkb_ref,
                 lq_ref, rq_ref, out_ref):
    rs = rs_ref[...].astype(_BF)
    rd = rd_ref[...].astype(_BF)
    rl = rl_ref[...].astype(_BF)
    l = (jnp.dot(rs, wqa_ref[...], preferred_element_type=jnp.float32)
         + jnp.dot(rl, wqb_ref[...], preferred_element_type=jnp.float32)
         + lq_ref[0])
    r = (jnp.dot(rd, wka_ref[...], preferred_element_type=jnp.float32)
         + jnp.dot(rl, wkb_ref[...], preferred_element_type=jnp.float32)
         + rq_ref[0])
    out_ref[0, 0, :] = jnp.sum(l * r, axis=1)


def _logits(rsrc, rdst, rel, wqat, wqbt, wkat, wkbt, lq, rq):
    eb = lambda i: (i, 0)
    qb = lambda i: (i // BPQ, 0, 0)
    full = lambda i: (0, 0)
    out = pl.pallas_call(
        _logits_body,
        grid=(NBLK,),
        in_specs=[
            pl.BlockSpec((EBLK, D), eb),
            pl.BlockSpec((EBLK, D), eb),
            pl.BlockSpec((EBLK, D), eb),
            pl.BlockSpec((D, 512), full),
            pl.BlockSpec((D, 512), full),
            pl.BlockSpec((D, 512), full),
            pl.BlockSpec((D, 512), full),
            pl.BlockSpec((1, 1, 512), qb),
            pl.BlockSpec((1, 1, 512), qb),
        ],
        out_specs=pl.BlockSpec((1, 1, EBLK), lambda i: (i, 0, 0)),
        out_shape=jax.ShapeDtypeStruct((NBLK, 1, EBLK), jnp.float32),
    )(rsrc, rdst, rel, wqat, wqbt, wkat, wkbt,
      lq.reshape(Q, 1, 512), rq.reshape(Q, 1, 512))
    return out.reshape(E)


def _finish_body(sa_ref, fa_ref, agg_ref, rep_ref, wl_ref, bl_ref,
                 score_ref, repr_ref):
    score_ref[...] = sa_ref[...]
    flags = fa_ref[...] > 0.0
    h = jnp.where(flags, agg_ref[...], rep_ref[...]).astype(_BF)
    y = (jnp.dot(h, wl_ref[...], preferred_element_type=jnp.float32)
         + bl_ref[...])
    repr_ref[...] = jnp.where(y > 0.0, y, 0.01 * y)


def _finish(score_sum, flag_sum, agg, rep, wlt, bl):
    RB = 2000
    G = N_NODES // RB
    rb2 = lambda i: (i, 0)
    full = lambda i: (0, 0)
    score2, repr_out = pl.pallas_call(
        _finish_body,
        grid=(G,),
        in_specs=[
            pl.BlockSpec((RB, 1), rb2),
            pl.BlockSpec((RB, 1), rb2),
            pl.BlockSpec((RB, D), rb2),
            pl.BlockSpec((RB, D), rb2),
            pl.BlockSpec((128, 128), full),
            pl.BlockSpec((1, 128), full),
        ],
        out_specs=[pl.BlockSpec((RB, 1), rb2), pl.BlockSpec((RB, D), rb2)],
        out_shape=[jax.ShapeDtypeStruct((N_NODES, 1), jnp.float32),
                   jax.ShapeDtypeStruct((N_NODES, D), jnp.float32)],
    )(score_sum.reshape(N_NODES, 1), flag_sum.reshape(N_NODES, 1),
      agg, rep, wlt, bl.reshape(1, 128))
    return score2.reshape(N_NODES), repr_out


def kernel(visited_node_score, visited_node_representation, rel_emb,
           query_src_ts_emb, query_rel_emb, Wq, Wk, Wl, bl,
           src, dst, query_idx):
    rep = visited_node_representation
    wqat = Wq[:, 0:128].T.astype(_BF)
    wqbt = Wq[:, 128:256].T.astype(_BF)
    wqct = Wq[:, 256:384].T.astype(_BF)
    wqdt = Wq[:, 384:512].T.astype(_BF)
    wkat = Wk[:, 0:128].T.astype(_BF)
    wkbt = Wk[:, 128:256].T.astype(_BF)
    wkct = Wk[:, 256:384].T.astype(_BF)
    wkdt = Wk[:, 384:512].T.astype(_BF)
    wlt = Wl.T.astype(_BF)

    lq, rq = _prep(query_src_ts_emb, query_rel_emb, wqct, wqdt, wkct, wkdt)

    rsrc = rep[src]
    rdst = rep[dst]

    logits = _logits(rsrc, rdst, rel_emb, wqat, wqbt, wkat, wkbt, lq, rq)

    p = jnp.exp(logits * (1.0 / 16.0))
    s1 = jnp.zeros((N_NODES,), jnp.float32).at[src].add(p)
    c = 16.0 * jnp.log(s1 + 1e-30)
    ex = jnp.exp(logits - c[src])
    denom = jnp.zeros((N_NODES,), jnp.float32).at[src].add(ex)
    soft = ex / denom[src]
    target = soft * visited_node_score[src]

    topv, topi = jax.lax.top_k(target.reshape(Q, EPQ), K)
    flat_idx = (topi + jnp.arange(Q, dtype=topi.dtype)[:, None] * EPQ).reshape(-1)
    p_src = src[flat_idx]
    p_dst = dst[flat_idx]
    p_soft = soft[flat_idx]
    p_tgt = target[flat_idx]

    score_sum = jnp.zeros((N_NODES,), jnp.float32).at[p_dst].add(p_tgt)
    flag_sum = jnp.zeros((N_NODES,), jnp.float32).at[p_src].add(1.0)
    msg = p_soft[:, None] * rep[p_dst]
    agg = jnp.zeros((N_NODES, D), jnp.float32).at[p_src].add(msg)

    return _finish(score_sum, flag_sum, agg, rep, wlt, bl)


# speedup vs baseline: 1.4883x; 1.1936x over previous
import functools
import jax
import jax.numpy as jnp
from jax import lax
from jax.experimental import pallas as pl
from jax.experimental.pallas import tpu as pltpu
from jax.experimental.pallas import tpu_sc as plsc

N_NODES = 50000
Q = 64
EPQ = 2048
E = Q * EPQ
D = 128
K = 128
EBLK = 512
NBLK = E // EBLK
BPQ = EPQ // EBLK

_BF = jnp.bfloat16


_NC = 2
_NS = 16
_NWK = _NC * _NS
_EPW = E // _NWK
_GW = 512
_GNW = _EPW // _GW


def _gather_body(rep_hbm, src_hbm, dst_hbm, osrc_hbm, odst_hbm,
                 idx_v, rows_v, sem):
    wid = lax.axis_index("s") * _NC + lax.axis_index("c")
    base = wid * _EPW
    for arr_hbm, out_hbm in ((src_hbm, osrc_hbm), (dst_hbm, odst_hbm)):
        for w in range(_GNW):
            off = base + w * _GW
            pltpu.sync_copy(arr_hbm.at[pl.ds(off, _GW)], idx_v)
            pltpu.async_copy(rep_hbm.at[idx_v], rows_v, sem).wait()
            pltpu.sync_copy(rows_v, out_hbm.at[pl.ds(off, _GW)])


def _gather_sc(rep, src, dst):
    f = pl.kernel(
        _gather_body,
        mesh=plsc.VectorSubcoreMesh(core_axis_name="c", subcore_axis_name="s"),
        out_type=[jax.ShapeDtypeStruct((E, D), jnp.float32),
                  jax.ShapeDtypeStruct((E, D), jnp.float32)],
        scratch_types=[pltpu.VMEM((_GW,), jnp.int32),
                       pltpu.VMEM((_GW, D), jnp.float32),
                       pltpu.SemaphoreType.DMA],
    )
    return f(rep, src, dst)


def _prep_body(qs_ref, qr_ref, wqc_ref, wqd_ref, wkc_ref, wkd_ref,
               lq_ref, rq_ref):
    qs = qs_ref[...].astype(_BF)
    qr = qr_ref[...].astype(_BF)
    lq_ref[...] = (jnp.dot(qs, wqc_ref[...], preferred_element_type=jnp.float32)
                   + jnp.dot(qr, wqd_ref[...], preferred_element_type=jnp.float32))
    rq_ref[...] = (jnp.dot(qs, wkc_ref[...], preferred_element_type=jnp.float32)
                   + jnp.dot(qr, wkd_ref[...], preferred_element_type=jnp.float32))


def _prep(qs, qr, wqct, wqdt, wkct, wkdt):
    outs = [jax.ShapeDtypeStruct((Q, 512), jnp.float32)] * 2
    return pl.pallas_call(_prep_body, out_shape=outs)(
        qs, qr, wqct, wqdt, wkct, wkdt)


def _logits_body(rs_ref, rd_ref, rl_ref, wqa_ref, wqb_ref, wka_ref, wkb_ref,
                 lq_ref, rq_ref, out_ref):
    rs = rs_ref[...].astype(_BF)
    rd = rd_ref[...].astype(_BF)
    rl = rl_ref[...].astype(_BF)
    l = (jnp.dot(rs, wqa_ref[...], preferred_element_type=jnp.float32)
         + jnp.dot(rl, wqb_ref[...], preferred_element_type=jnp.float32)
         + lq_ref[0])
    r = (jnp.dot(rd, wka_ref[...], preferred_element_type=jnp.float32)
         + jnp.dot(rl, wkb_ref[...], preferred_element_type=jnp.float32)
         + rq_ref[0])
    out_ref[0, 0, :] = jnp.sum(l * r, axis=1)


def _logits(rsrc, rdst, rel, wqat, wqbt, wkat, wkbt, lq, rq):
    eb = lambda i: (i, 0)
    qb = lambda i: (i // BPQ, 0, 0)
    full = lambda i: (0, 0)
    out = pl.pallas_call(
        _logits_body,
        grid=(NBLK,),
        in_specs=[
            pl.BlockSpec((EBLK, D), eb),
            pl.BlockSpec((EBLK, D), eb),
            pl.BlockSpec((EBLK, D), eb),
            pl.BlockSpec((D, 512), full),
            pl.BlockSpec((D, 512), full),
            pl.BlockSpec((D, 512), full),
            pl.BlockSpec((D, 512), full),
            pl.BlockSpec((1, 1, 512), qb),
            pl.BlockSpec((1, 1, 512), qb),
        ],
        out_specs=pl.BlockSpec((1, 1, EBLK), lambda i: (i, 0, 0)),
        out_shape=jax.ShapeDtypeStruct((NBLK, 1, EBLK), jnp.float32),
    )(rsrc, rdst, rel, wqat, wqbt, wkat, wkbt,
      lq.reshape(Q, 1, 512), rq.reshape(Q, 1, 512))
    return out.reshape(E)


def _finish_body(sa_ref, fa_ref, agg_ref, rep_ref, wl_ref, bl_ref,
                 score_ref, repr_ref):
    score_ref[...] = sa_ref[...]
    flags = fa_ref[...] > 0.0
    h = jnp.where(flags, agg_ref[...], rep_ref[...]).astype(_BF)
    y = (jnp.dot(h, wl_ref[...], preferred_element_type=jnp.float32)
         + bl_ref[...])
    repr_ref[...] = jnp.where(y > 0.0, y, 0.01 * y)


def _finish(score_sum, flag_sum, agg, rep, wlt, bl):
    RB = 2000
    G = N_NODES // RB
    rb2 = lambda i: (i, 0)
    full = lambda i: (0, 0)
    score2, repr_out = pl.pallas_call(
        _finish_body,
        grid=(G,),
        in_specs=[
            pl.BlockSpec((RB, 1), rb2),
            pl.BlockSpec((RB, 1), rb2),
            pl.BlockSpec((RB, D), rb2),
            pl.BlockSpec((RB, D), rb2),
            pl.BlockSpec((128, 128), full),
            pl.BlockSpec((1, 128), full),
        ],
        out_specs=[pl.BlockSpec((RB, 1), rb2), pl.BlockSpec((RB, D), rb2)],
        out_shape=[jax.ShapeDtypeStruct((N_NODES, 1), jnp.float32),
                   jax.ShapeDtypeStruct((N_NODES, D), jnp.float32)],
    )(score_sum.reshape(N_NODES, 1), flag_sum.reshape(N_NODES, 1),
      agg, rep, wlt, bl.reshape(1, 128))
    return score2.reshape(N_NODES), repr_out


def kernel(visited_node_score, visited_node_representation, rel_emb,
           query_src_ts_emb, query_rel_emb, Wq, Wk, Wl, bl,
           src, dst, query_idx):
    rep = visited_node_representation
    wqat = Wq[:, 0:128].T.astype(_BF)
    wqbt = Wq[:, 128:256].T.astype(_BF)
    wqct = Wq[:, 256:384].T.astype(_BF)
    wqdt = Wq[:, 384:512].T.astype(_BF)
    wkat = Wk[:, 0:128].T.astype(_BF)
    wkbt = Wk[:, 128:256].T.astype(_BF)
    wkct = Wk[:, 256:384].T.astype(_BF)
    wkdt = Wk[:, 384:512].T.astype(_BF)
    wlt = Wl.T.astype(_BF)

    lq, rq = _prep(query_src_ts_emb, query_rel_emb, wqct, wqdt, wkct, wkdt)

    rsrc, rdst = _gather_sc(rep, src, dst)

    logits = _logits(rsrc, rdst, rel_emb, wqat, wqbt, wkat, wkbt, lq, rq)

    p = jnp.exp(logits * (1.0 / 16.0))
    s1 = jnp.zeros((N_NODES,), jnp.float32).at[src].add(p)
    c = 16.0 * jnp.log(s1 + 1e-30)
    ex = jnp.exp(logits - c[src])
    denom = jnp.zeros((N_NODES,), jnp.float32).at[src].add(ex)
    soft = ex / denom[src]
    target = soft * visited_node_score[src]

    topv, topi = jax.lax.top_k(target.reshape(Q, EPQ), K)
    flat_idx = (topi + jnp.arange(Q, dtype=topi.dtype)[:, None] * EPQ).reshape(-1)
    p_src = src[flat_idx]
    p_dst = dst[flat_idx]
    p_soft = soft[flat_idx]
    p_tgt = target[flat_idx]

    score_sum = jnp.zeros((N_NODES,), jnp.float32).at[p_dst].add(p_tgt)
    flag_sum = jnp.zeros((N_NODES,), jnp.float32).at[p_src].add(1.0)
    msg = p_soft[:, None] * rep[p_dst]
    agg = jnp.zeros((N_NODES, D), jnp.float32).at[p_src].add(msg)

    return _finish(score_sum, flag_sum, agg, rep, wlt, bl)


# speedup vs baseline: 7.0625x; 4.7453x over previous
import functools
import jax
import jax.numpy as jnp
from jax import lax
from jax.experimental import pallas as pl
from jax.experimental.pallas import tpu as pltpu
from jax.experimental.pallas import tpu_sc as plsc

N_NODES = 50000
Q = 64
EPQ = 2048
E = Q * EPQ
D = 128
K = 128
EBLK = 512
NBLK = E // EBLK
BPQ = EPQ // EBLK

_BF = jnp.bfloat16


_NC = 2
_NS = 16
_NWK = _NC * _NS
_EPW = E // _NWK
_GW = 512
_GNW = _EPW // _GW


def _gather_body(rep_hbm, src_hbm, dst_hbm, osrc_hbm, odst_hbm,
                 idx_v, rows_v, sem):
    wid = lax.axis_index("s") * _NC + lax.axis_index("c")
    base = wid * _EPW
    for arr_hbm, out_hbm in ((src_hbm, osrc_hbm), (dst_hbm, odst_hbm)):
        for w in range(_GNW):
            off = base + w * _GW
            pltpu.sync_copy(arr_hbm.at[pl.ds(off, _GW)], idx_v)
            pltpu.async_copy(rep_hbm.at[idx_v], rows_v, sem).wait()
            pltpu.sync_copy(rows_v, out_hbm.at[pl.ds(off, _GW)])


def _gather_sc(rep, src, dst):
    f = pl.kernel(
        _gather_body,
        mesh=plsc.VectorSubcoreMesh(core_axis_name="c", subcore_axis_name="s"),
        out_type=[jax.ShapeDtypeStruct((E, D), jnp.float32),
                  jax.ShapeDtypeStruct((E, D), jnp.float32)],
        scratch_types=[pltpu.VMEM((_GW,), jnp.int32),
                       pltpu.VMEM((_GW, D), jnp.float32),
                       pltpu.SemaphoreType.DMA],
    )
    return f(rep, src, dst)


_NPAD = 51200
_TSL = _NPAD // _NS
_ER = E // 128
_RPW = _ER // _NWK


def _zero_slice(buf, shared, sid):
    def zb(i, _):
        buf[pl.ds(i * 16, 16)] = jnp.zeros((16,), jnp.float32)
        return 0
    lax.fori_loop(0, _TSL // 16, zb, 0)
    pltpu.sync_copy(buf, shared.at[pl.ds(sid * _TSL, _TSL)])


def _combine_tables(tab_hbm, table_v, stage):
    pltpu.sync_copy(tab_hbm.at[pl.ds(0, _NPAD)], table_v)
    for t in range(_NS):
        pltpu.sync_copy(tab_hbm.at[pl.ds(_NPAD + t * _TSL, _TSL)], stage)

        def ab(i, _):
            o = t * _TSL + i * 16
            table_v[pl.ds(o, 16)] = table_v[pl.ds(o, 16)] + stage[pl.ds(i * 16, 16)]
            return 0
        lax.fori_loop(0, _TSL // 16, ab, 0)


def _sm1_body(l_hbm, src_hbm, p_hbm, stab_hbm, zbuf, l2d, sv2d, p2d, shared):
    cid = lax.axis_index("c")
    sid = lax.axis_index("s")
    wid = sid * _NC + cid
    rowbase = wid * _RPW
    _zero_slice(zbuf, shared, sid)
    pltpu.sync_copy(l_hbm.at[pl.ds(rowbase, _RPW)], l2d)
    pltpu.sync_copy(src_hbm.at[pl.ds(rowbase, _RPW)], sv2d)

    for j in range(_RPW):
        for k in range(8):
            lv = l2d[j, pl.ds(k * 16, 16)]
            p2d[j, pl.ds(k * 16, 16)] = jnp.exp(lv * 0.0625)
    pltpu.sync_copy(p2d, p_hbm.at[pl.ds(rowbase, _RPW)])
    plsc.subcore_barrier()
    for j in range(_RPW):
        pltpu.sync_copy(p2d.at[j], shared.at[sv2d.at[j]], add=True)
    plsc.subcore_barrier()
    off = pl.multiple_of(cid * _NPAD + sid * _TSL, 8)
    pltpu.sync_copy(shared.at[pl.ds(sid * _TSL, _TSL)], zbuf)
    pltpu.sync_copy(zbuf, stab_hbm.at[pl.ds(off, _TSL)])


def _sm2_body(p_hbm, src_hbm, stab_hbm, ex_hbm, dtab_hbm,
              bufa, bufb, sv2d, p2d, ex2d, g2d, shared_acc, shared_tab):
    cid = lax.axis_index("c")
    sid = lax.axis_index("s")
    wid = sid * _NC + cid
    rowbase = wid * _RPW
    off = pl.multiple_of(sid * _TSL, 8)
    pltpu.sync_copy(stab_hbm.at[pl.ds(off, _TSL)], bufa)
    pltpu.sync_copy(stab_hbm.at[pl.ds(_NPAD + off, _TSL)], bufb)

    def ab(i, _):
        bufa[pl.ds(i * 16, 16)] = bufa[pl.ds(i * 16, 16)] + bufb[pl.ds(i * 16, 16)]
        return 0
    lax.fori_loop(0, _TSL // 16, ab, 0)
    pltpu.sync_copy(bufa, shared_tab.at[pl.ds(off, _TSL)])

    def zb(i, _):
        bufb[pl.ds(i * 16, 16)] = jnp.zeros((16,), jnp.float32)
        return 0
    lax.fori_loop(0, _TSL // 16, zb, 0)
    pltpu.sync_copy(bufb, shared_acc.at[pl.ds(off, _TSL)])
    pltpu.sync_copy(p_hbm.at[pl.ds(rowbase, _RPW)], p2d)
    pltpu.sync_copy(src_hbm.at[pl.ds(rowbase, _RPW)], sv2d)
    plsc.subcore_barrier()
    for j in range(_RPW):
        pltpu.sync_copy(shared_tab.at[sv2d.at[j]], g2d.at[j])
    for j in range(_RPW):
        for k in range(8):
            r = p2d[j, pl.ds(k * 16, 16)] / g2d[j, pl.ds(k * 16, 16)]
            r = r * r
            r = r * r
            r = r * r
            ex2d[j, pl.ds(k * 16, 16)] = r * r
    pltpu.sync_copy(ex2d, ex_hbm.at[pl.ds(rowbase, _RPW)])
    for j in range(_RPW):
        pltpu.sync_copy(ex2d.at[j], shared_acc.at[sv2d.at[j]], add=True)
    plsc.subcore_barrier()
    pltpu.sync_copy(shared_acc.at[pl.ds(off, _TSL)], bufa)
    oo = pl.multiple_of(cid * _NPAD + sid * _TSL, 8)
    pltpu.sync_copy(bufa, dtab_hbm.at[pl.ds(oo, _TSL)])


def _sm3_body(ex_hbm, src_hbm, dtab_hbm, scorep_hbm, soft_hbm, tgt_hbm,
              bufa, bufb, sv2d, ex2d, g2d, so2d, tg2d, shared_dt, shared_sc):
    cid = lax.axis_index("c")
    sid = lax.axis_index("s")
    wid = sid * _NC + cid
    rowbase = wid * _RPW
    off = pl.multiple_of(sid * _TSL, 8)
    pltpu.sync_copy(dtab_hbm.at[pl.ds(off, _TSL)], bufa)
    pltpu.sync_copy(dtab_hbm.at[pl.ds(_NPAD + off, _TSL)], bufb)

    def ab(i, _):
        bufa[pl.ds(i * 16, 16)] = bufa[pl.ds(i * 16, 16)] + bufb[pl.ds(i * 16, 16)]
        return 0
    lax.fori_loop(0, _TSL // 16, ab, 0)
    pltpu.sync_copy(bufa, shared_dt.at[pl.ds(off, _TSL)])
    pltpu.sync_copy(scorep_hbm.at[pl.ds(off, _TSL)], bufb)
    pltpu.sync_copy(bufb, shared_sc.at[pl.ds(off, _TSL)])
    pltpu.sync_copy(ex_hbm.at[pl.ds(rowbase, _RPW)], ex2d)
    pltpu.sync_copy(src_hbm.at[pl.ds(rowbase, _RPW)], sv2d)
    plsc.subcore_barrier()
    for j in range(_RPW):
        pltpu.sync_copy(shared_dt.at[sv2d.at[j]], g2d.at[j])
    for j in range(_RPW):
        for k in range(8):
            so2d[j, pl.ds(k * 16, 16)] = (ex2d[j, pl.ds(k * 16, 16)]
                                          / g2d[j, pl.ds(k * 16, 16)])
    for j in range(_RPW):
        pltpu.sync_copy(shared_sc.at[sv2d.at[j]], g2d.at[j])
    for j in range(_RPW):
        for k in range(8):
            tg2d[j, pl.ds(k * 16, 16)] = (so2d[j, pl.ds(k * 16, 16)]
                                          * g2d[j, pl.ds(k * 16, 16)])
    pltpu.sync_copy(so2d, soft_hbm.at[pl.ds(rowbase, _RPW)])
    pltpu.sync_copy(tg2d, tgt_hbm.at[pl.ds(rowbase, _RPW)])


def _softmax_sc(logits2d, src2d, score_pad):
    mesh = plsc.VectorSubcoreMesh(core_axis_name="c", subcore_axis_name="s")
    f32 = jnp.float32
    k1 = pl.kernel(
        _sm1_body, mesh=mesh,
        out_type=[jax.ShapeDtypeStruct((_ER, 128), f32),
                  jax.ShapeDtypeStruct((2 * _NPAD,), f32)],
        scratch_types=[pltpu.VMEM((_TSL,), f32),
                       pltpu.VMEM((_RPW, 128), f32),
                       pltpu.VMEM((_RPW, 128), jnp.int32),
                       pltpu.VMEM((_RPW, 128), f32),
                       pltpu.VMEM_SHARED((_NPAD,), f32)],
    )
    p2d, stab = k1(logits2d, src2d)
    k2 = pl.kernel(
        _sm2_body, mesh=mesh,
        out_type=[jax.ShapeDtypeStruct((_ER, 128), f32),
                  jax.ShapeDtypeStruct((2 * _NPAD,), f32)],
        scratch_types=[pltpu.VMEM((_TSL,), f32),
                       pltpu.VMEM((_TSL,), f32),
                       pltpu.VMEM((_RPW, 128), jnp.int32),
                       pltpu.VMEM((_RPW, 128), f32),
                       pltpu.VMEM((_RPW, 128), f32),
                       pltpu.VMEM((_RPW, 128), f32),
                       pltpu.VMEM_SHARED((_NPAD,), f32),
                       pltpu.VMEM_SHARED((_NPAD,), f32)],
    )
    ex2d, dtab = k2(p2d, src2d, stab)
    k3 = pl.kernel(
        _sm3_body, mesh=mesh,
        out_type=[jax.ShapeDtypeStruct((_ER, 128), f32),
                  jax.ShapeDtypeStruct((_ER, 128), f32)],
        scratch_types=[pltpu.VMEM((_TSL,), f32),
                       pltpu.VMEM((_TSL,), f32),
                       pltpu.VMEM((_RPW, 128), jnp.int32),
                       pltpu.VMEM((_RPW, 128), f32),
                       pltpu.VMEM((_RPW, 128), f32),
                       pltpu.VMEM((_RPW, 128), f32),
                       pltpu.VMEM((_RPW, 128), f32),
                       pltpu.VMEM_SHARED((_NPAD,), f32),
                       pltpu.VMEM_SHARED((_NPAD,), f32)],
    )
    soft2d, tgt2d = k3(ex2d, src2d, dtab, score_pad)
    return soft2d.reshape(E), tgt2d.reshape(E)


def _prep_body(qs_ref, qr_ref, wqc_ref, wqd_ref, wkc_ref, wkd_ref,
               lq_ref, rq_ref):
    qs = qs_ref[...].astype(_BF)
    qr = qr_ref[...].astype(_BF)
    lq_ref[...] = (jnp.dot(qs, wqc_ref[...], preferred_element_type=jnp.float32)
                   + jnp.dot(qr, wqd_ref[...], preferred_element_type=jnp.float32))
    rq_ref[...] = (jnp.dot(qs, wkc_ref[...], preferred_element_type=jnp.float32)
                   + jnp.dot(qr, wkd_ref[...], preferred_element_type=jnp.float32))


def _prep(qs, qr, wqct, wqdt, wkct, wkdt):
    outs = [jax.ShapeDtypeStruct((Q, 512), jnp.float32)] * 2
    return pl.pallas_call(_prep_body, out_shape=outs)(
        qs, qr, wqct, wqdt, wkct, wkdt)


def _logits_body(rs_ref, rd_ref, rl_ref, wqa_ref, wqb_ref, wka_ref, wkb_ref,
                 lq_ref, rq_ref, out_ref):
    rs = rs_ref[...].astype(_BF)
    rd = rd_ref[...].astype(_BF)
    rl = rl_ref[...].astype(_BF)
    l = (jnp.dot(rs, wqa_ref[...], preferred_element_type=jnp.float32)
         + jnp.dot(rl, wqb_ref[...], preferred_element_type=jnp.float32)
         + lq_ref[0])
    r = (jnp.dot(rd, wka_ref[...], preferred_element_type=jnp.float32)
         + jnp.dot(rl, wkb_ref[...], preferred_element_type=jnp.float32)
         + rq_ref[0])
    out_ref[0, 0, :] = jnp.sum(l * r, axis=1)


def _logits(rsrc, rdst, rel, wqat, wqbt, wkat, wkbt, lq, rq):
    eb = lambda i: (i, 0)
    qb = lambda i: (i // BPQ, 0, 0)
    full = lambda i: (0, 0)
    out = pl.pallas_call(
        _logits_body,
        grid=(NBLK,),
        in_specs=[
            pl.BlockSpec((EBLK, D), eb),
            pl.BlockSpec((EBLK, D), eb),
            pl.BlockSpec((EBLK, D), eb),
            pl.BlockSpec((D, 512), full),
            pl.BlockSpec((D, 512), full),
            pl.BlockSpec((D, 512), full),
            pl.BlockSpec((D, 512), full),
            pl.BlockSpec((1, 1, 512), qb),
            pl.BlockSpec((1, 1, 512), qb),
        ],
        out_specs=pl.BlockSpec((1, 1, EBLK), lambda i: (i, 0, 0)),
        out_shape=jax.ShapeDtypeStruct((NBLK, 1, EBLK), jnp.float32),
    )(rsrc, rdst, rel, wqat, wqbt, wkat, wkbt,
      lq.reshape(Q, 1, 512), rq.reshape(Q, 1, 512))
    return out.reshape(E)


def _finish_body(sa_ref, fa_ref, agg_ref, rep_ref, wl_ref, bl_ref,
                 score_ref, repr_ref):
    score_ref[...] = sa_ref[...]
    flags = fa_ref[...] > 0.0
    h = jnp.where(flags, agg_ref[...], rep_ref[...]).astype(_BF)
    y = (jnp.dot(h, wl_ref[...], preferred_element_type=jnp.float32)
         + bl_ref[...])
    repr_ref[...] = jnp.where(y > 0.0, y, 0.01 * y)


def _finish(score_sum, flag_sum, agg, rep, wlt, bl):
    RB = 2000
    G = N_NODES // RB
    rb2 = lambda i: (i, 0)
    full = lambda i: (0, 0)
    score2, repr_out = pl.pallas_call(
        _finish_body,
        grid=(G,),
        in_specs=[
            pl.BlockSpec((RB, 1), rb2),
            pl.BlockSpec((RB, 1), rb2),
            pl.BlockSpec((RB, D), rb2),
            pl.BlockSpec((RB, D), rb2),
            pl.BlockSpec((128, 128), full),
            pl.BlockSpec((1, 128), full),
        ],
        out_specs=[pl.BlockSpec((RB, 1), rb2), pl.BlockSpec((RB, D), rb2)],
        out_shape=[jax.ShapeDtypeStruct((N_NODES, 1), jnp.float32),
                   jax.ShapeDtypeStruct((N_NODES, D), jnp.float32)],
    )(score_sum.reshape(N_NODES, 1), flag_sum.reshape(N_NODES, 1),
      agg, rep, wlt, bl.reshape(1, 128))
    return score2.reshape(N_NODES), repr_out


def kernel(visited_node_score, visited_node_representation, rel_emb,
           query_src_ts_emb, query_rel_emb, Wq, Wk, Wl, bl,
           src, dst, query_idx):
    rep = visited_node_representation
    wqat = Wq[:, 0:128].T.astype(_BF)
    wqbt = Wq[:, 128:256].T.astype(_BF)
    wqct = Wq[:, 256:384].T.astype(_BF)
    wqdt = Wq[:, 384:512].T.astype(_BF)
    wkat = Wk[:, 0:128].T.astype(_BF)
    wkbt = Wk[:, 128:256].T.astype(_BF)
    wkct = Wk[:, 256:384].T.astype(_BF)
    wkdt = Wk[:, 384:512].T.astype(_BF)
    wlt = Wl.T.astype(_BF)

    lq, rq = _prep(query_src_ts_emb, query_rel_emb, wqct, wqdt, wkct, wkdt)

    rsrc, rdst = _gather_sc(rep, src, dst)

    logits = _logits(rsrc, rdst, rel_emb, wqat, wqbt, wkat, wkbt, lq, rq)

    logits2d = logits.reshape(_ER, 128)
    src2d = src.reshape(_ER, 128)
    score_pad = jnp.pad(visited_node_score, (0, _NPAD - N_NODES))
    soft, target = _softmax_sc(logits2d, src2d, score_pad)

    topv, topi = jax.lax.top_k(target.reshape(Q, EPQ), K)
    flat_idx = (topi + jnp.arange(Q, dtype=topi.dtype)[:, None] * EPQ).reshape(-1)
    p_src = src[flat_idx]
    p_dst = dst[flat_idx]
    p_soft = soft[flat_idx]
    p_tgt = target[flat_idx]

    score_sum = jnp.zeros((N_NODES,), jnp.float32).at[p_dst].add(p_tgt)
    flag_sum = jnp.zeros((N_NODES,), jnp.float32).at[p_src].add(1.0)
    msg = p_soft[:, None] * rep[p_dst]
    agg = jnp.zeros((N_NODES, D), jnp.float32).at[p_src].add(msg)

    return _finish(score_sum, flag_sum, agg, rep, wlt, bl)
